# Initial kernel scaffold; baseline (speedup 1.0000x reference)
import jax
import jax.numpy as jnp
from jax.experimental import pallas as pl


def kernel(x, w_win_qkv, b_win_qkv, w_bra_qkv, b_bra_qkv, lepe_w, lepe_b, ca_qkv_w, ca_proj_w, ca_proj_b):
    raise NotImplementedError("write your pallas kernel here")


# speedup vs baseline: 2.6018x; 2.6018x over previous
import jax
import jax.numpy as jnp
from jax import lax
from jax.experimental import pallas as pl
from jax.experimental.pallas import tpu as pltpu

DIM = 192
NH = 8
WS = 8
TOPK = 4
H = 224
NWH = H // WS
NREG = NWH * NWH
TOK = WS * WS
NTOK = NREG * TOK
CH = DIM // 2
HDW = CH // NH
HDC = DIM // NH
SCALE = DIM ** -0.5
CA_SCALE = HDC ** -0.5

_INTERPRET = False


def _qkv_kernel(x_ref, w_ref, b_ref, ow_ref, oq_ref, ok_ref, ov_ref, ol_ref):
    y = jnp.dot(x_ref[...], w_ref[...], preferred_element_type=jnp.float32)
    y = y + b_ref[...]
    ow_ref[...] = y[:, 0:288]
    oq_ref[...] = y[:, 288:384]
    ok_ref[...] = y[:, 384:480]
    ov_ref[...] = y[:, 480:576]
    ol_ref[...] = jnp.concatenate([y[:, 480:576], y[:, 192:288]], axis=1)


def _win_attn_kernel(qkv_ref, o_ref, *, wb):
    for w in range(wb):
        q = qkv_ref[w, :, 0:CH] * SCALE
        k = qkv_ref[w, :, CH:2 * CH]
        v = qkv_ref[w, :, 2 * CH:3 * CH]
        outs = []
        for h in range(NH):
            qh = q[:, h * HDW:(h + 1) * HDW]
            kh = k[:, h * HDW:(h + 1) * HDW]
            vh = v[:, h * HDW:(h + 1) * HDW]
            a = lax.dot_general(qh, kh, (((1,), (1,)), ((), ())),
                                preferred_element_type=jnp.float32)
            a = jax.nn.softmax(a, axis=-1)
            outs.append(jnp.dot(a, vh, preferred_element_type=jnp.float32))
        o_ref[w] = jnp.concatenate(outs, axis=1)


def _pool_kernel(q_ref, k_ref, qr_ref, kr_ref):
    qr_ref[...] = jnp.mean(q_ref[...], axis=1)
    kr_ref[...] = jnp.mean(k_ref[...], axis=1)


def _topk_kernel(qr_ref, kr_ref, idx_ref):
    a = lax.dot_general(qr_ref[...], kr_ref[...], (((1,), (1,)), ((), ())),
                        preferred_element_type=jnp.float32)
    col = lax.broadcasted_iota(jnp.int32, a.shape, 1)
    idxs = []
    for _ in range(TOPK):
        m = jnp.max(a, axis=1, keepdims=True)
        sel = jnp.where(a >= m, col, jnp.int32(2 ** 30))
        j = jnp.min(sel, axis=1, keepdims=True)
        idxs.append(j)
        a = jnp.where(col == j, jnp.float32(-3e38), a)
    idx_ref[...] = jnp.concatenate(idxs, axis=1)


def _bra_attn_kernel(idx_ref, q_ref, k0_ref, k1_ref, k2_ref, k3_ref,
                     v0_ref, v1_ref, v2_ref, v3_ref, o_ref):
    del idx_ref
    q = q_ref[0] * SCALE
    kc = jnp.concatenate([k0_ref[0], k1_ref[0], k2_ref[0], k3_ref[0]], axis=0)
    vc = jnp.concatenate([v0_ref[0], v1_ref[0], v2_ref[0], v3_ref[0]], axis=0)
    outs = []
    for h in range(NH):
        qh = q[:, h * HDW:(h + 1) * HDW]
        kh = kc[:, h * HDW:(h + 1) * HDW]
        vh = vc[:, h * HDW:(h + 1) * HDW]
        a = lax.dot_general(qh, kh, (((1,), (1,)), ((), ())),
                            preferred_element_type=jnp.float32)
        a = jax.nn.softmax(a, axis=-1)
        outs.append(jnp.dot(a, vh, preferred_element_type=jnp.float32))
    o_ref[0] = jnp.concatenate(outs, axis=1)


def _lepe_kernel(x0_ref, x1_ref, x2_ref, w_ref, b_ref, o_ref):
    acc = jnp.broadcast_to(b_ref[...], (8, H, DIM)).astype(jnp.float32)
    for dy, xref in enumerate((x0_ref, x1_ref, x2_ref)):
        xfull = xref[...]
        for dx in range(3):
            wrow = w_ref[dy * 3 + dx, :]
            acc = acc + xfull[:, dx:dx + H, :] * wrow[None, None, :]
    o_ref[...] = acc


def _ca1_kernel(win_ref, bra_ref, lepe_ref, w_ref, q_ref, kv_ref):
    x = jnp.concatenate([win_ref[...], bra_ref[...]], axis=1) + lepe_ref[...]
    qkv = jnp.dot(x, w_ref[...], preferred_element_type=jnp.float32)
    q_ref[...] = qkv[:, 0:DIM]
    k = qkv[:, DIM:2 * DIM] * CA_SCALE
    v = qkv[:, 2 * DIM:3 * DIM]

    @pl.when(pl.program_id(0) == 0)
    def _():
        kv_ref[...] = jnp.zeros_like(kv_ref)

    kvs = []
    for h in range(NH):
        kh = k[:, h * HDC:(h + 1) * HDC]
        vh = v[:, h * HDC:(h + 1) * HDC]
        kvs.append(lax.dot_general(kh, vh, (((0,), (0,)), ((), ())),
                                   preferred_element_type=jnp.float32))
    kv_ref[...] += jnp.concatenate(kvs, axis=0)


def _ca2_kernel(kv_ref, q_ref, pw_ref, pb_ref, o_ref):
    outs = []
    for h in range(NH):
        logits = kv_ref[h * HDC:(h + 1) * HDC, :]
        att = jax.nn.softmax(logits, axis=-1)
        qh = q_ref[:, h * HDC:(h + 1) * HDC]
        outs.append(lax.dot_general(qh, att, (((1,), (1,)), ((), ())),
                                    preferred_element_type=jnp.float32))
    o = jnp.concatenate(outs, axis=1)
    o_ref[...] = jnp.dot(o, pw_ref[...], preferred_element_type=jnp.float32) + pb_ref[...]


def _to_windows(t):
    c = t.shape[-1]
    t = t.reshape(NWH, WS, NWH, WS, c).transpose(0, 2, 1, 3, 4)
    return t.reshape(NREG, TOK, c)


def _from_windows(t):
    c = t.shape[-1]
    t = t.reshape(NWH, NWH, WS, WS, c).transpose(0, 2, 1, 3, 4)
    return t.reshape(NTOK, c)


def kernel(x, w_win_qkv, b_win_qkv, w_bra_qkv, b_bra_qkv, lepe_w, lepe_b,
           ca_qkv_w, ca_proj_w, ca_proj_b):
    f32 = jnp.float32
    xw = _to_windows(x)
    xf = xw.reshape(NTOK, DIM)

    z = jnp.zeros((CH, 3 * CH), f32)
    wc = jnp.concatenate([jnp.concatenate([w_win_qkv, z], axis=1),
                          jnp.concatenate([z, w_bra_qkv], axis=1)], axis=0)
    bc = jnp.concatenate([b_win_qkv, b_bra_qkv])[None, :]

    rb = 1792
    ngrid = NTOK // rb
    o_win, o_qb, o_kb, o_vb, o_lep = pl.pallas_call(
        _qkv_kernel,
        grid=(ngrid,),
        in_specs=[
            pl.BlockSpec((rb, DIM), lambda i: (i, 0)),
            pl.BlockSpec((DIM, 576), lambda i: (0, 0)),
            pl.BlockSpec((1, 576), lambda i: (0, 0)),
        ],
        out_specs=[
            pl.BlockSpec((rb, 288), lambda i: (i, 0)),
            pl.BlockSpec((rb, 96), lambda i: (i, 0)),
            pl.BlockSpec((rb, 96), lambda i: (i, 0)),
            pl.BlockSpec((rb, 96), lambda i: (i, 0)),
            pl.BlockSpec((rb, 192), lambda i: (i, 0)),
        ],
        out_shape=[
            jax.ShapeDtypeStruct((NTOK, 288), f32),
            jax.ShapeDtypeStruct((NTOK, 96), f32),
            jax.ShapeDtypeStruct((NTOK, 96), f32),
            jax.ShapeDtypeStruct((NTOK, 96), f32),
            jax.ShapeDtypeStruct((NTOK, 192), f32),
        ],
        interpret=_INTERPRET,
    )(xf, wc, bc)

    qkv_win = o_win.reshape(NREG, TOK, 288)
    q_bra = o_qb.reshape(NREG, TOK, 96)
    k_bra = o_kb.reshape(NREG, TOK, 96)
    v_bra = o_vb.reshape(NREG, TOK, 96)

    wb = 8
    import functools
    win_out = pl.pallas_call(
        functools.partial(_win_attn_kernel, wb=wb),
        grid=(NREG // wb,),
        in_specs=[pl.BlockSpec((wb, TOK, 288), lambda i: (i, 0, 0))],
        out_specs=pl.BlockSpec((wb, TOK, 96), lambda i: (i, 0, 0)),
        out_shape=jax.ShapeDtypeStruct((NREG, TOK, 96), f32),
        interpret=_INTERPRET,
    )(qkv_win)

    pb_ = 112
    q_r, k_r = pl.pallas_call(
        _pool_kernel,
        grid=(NREG // pb_,),
        in_specs=[pl.BlockSpec((pb_, TOK, 96), lambda i: (i, 0, 0)),
                  pl.BlockSpec((pb_, TOK, 96), lambda i: (i, 0, 0))],
        out_specs=[pl.BlockSpec((pb_, 96), lambda i: (i, 0)),
                   pl.BlockSpec((pb_, 96), lambda i: (i, 0))],
        out_shape=[jax.ShapeDtypeStruct((NREG, 96), f32),
                   jax.ShapeDtypeStruct((NREG, 96), f32)],
        interpret=_INTERPRET,
    )(q_bra, k_bra)

    idx = pl.pallas_call(
        _topk_kernel,
        grid=(NREG // pb_,),
        in_specs=[pl.BlockSpec((pb_, 96), lambda i: (i, 0)),
                  pl.BlockSpec((NREG, 96), lambda i: (0, 0))],
        out_specs=pl.BlockSpec((pb_, TOPK), lambda i: (i, 0)),
        out_shape=jax.ShapeDtypeStruct((NREG, TOPK), jnp.int32),
        interpret=_INTERPRET,
    )(q_r, k_r)
    idx_flat = idx.reshape(-1)

    def _qmap(i, idx_ref):
        return (i, 0, 0)

    def _gmap(t):
        def m(i, idx_ref):
            return (idx_ref[i * TOPK + t], 0, 0)
        return m

    bra_out = pl.pallas_call(
        _bra_attn_kernel,
        grid_spec=pltpu.PrefetchScalarGridSpec(
            num_scalar_prefetch=1,
            grid=(NREG,),
            in_specs=[pl.BlockSpec((1, TOK, 96), _qmap)]
                     + [pl.BlockSpec((1, TOK, 96), _gmap(t)) for t in range(TOPK)]
                     + [pl.BlockSpec((1, TOK, 96), _gmap(t)) for t in range(TOPK)],
            out_specs=pl.BlockSpec((1, TOK, 96), _qmap),
        ),
        out_shape=jax.ShapeDtypeStruct((NREG, TOK, 96), f32),
        interpret=_INTERPRET,
    )(idx_flat, q_bra, k_bra, k_bra, k_bra, k_bra,
      v_bra, v_bra, v_bra, v_bra)

    lepe_hwc = _from_windows(o_lep.reshape(NREG, TOK, 192)).reshape(H, H, DIM)
    xp = jnp.pad(lepe_hwc, ((1, 1), (1, 1), (0, 0)))
    x0, x1, x2 = xp[0:H], xp[1:H + 1], xp[2:H + 2]
    wt = jnp.pad(lepe_w.reshape(DIM, 9).T, ((0, 7), (0, 0)))
    lb = lepe_b[None, :]
    lepe = pl.pallas_call(
        _lepe_kernel,
        grid=(H // 8,),
        in_specs=[pl.BlockSpec((8, H + 2, DIM), lambda i: (i, 0, 0)),
                  pl.BlockSpec((8, H + 2, DIM), lambda i: (i, 0, 0)),
                  pl.BlockSpec((8, H + 2, DIM), lambda i: (i, 0, 0)),
                  pl.BlockSpec((16, DIM), lambda i: (0, 0)),
                  pl.BlockSpec((1, DIM), lambda i: (0, 0))],
        out_specs=pl.BlockSpec((8, H, DIM), lambda i: (i, 0, 0)),
        out_shape=jax.ShapeDtypeStruct((H, H, DIM), f32),
        interpret=_INTERPRET,
    )(x0, x1, x2, wt, lb)
    lepe_tok = lepe.reshape(NTOK, DIM)

    win_tok = _from_windows(win_out)
    bra_tok = _from_windows(bra_out)

    q_ca, kv = pl.pallas_call(
        _ca1_kernel,
        grid=(ngrid,),
        in_specs=[pl.BlockSpec((rb, 96), lambda i: (i, 0)),
                  pl.BlockSpec((rb, 96), lambda i: (i, 0)),
                  pl.BlockSpec((rb, DIM), lambda i: (i, 0)),
                  pl.BlockSpec((DIM, 576), lambda i: (0, 0))],
        out_specs=[pl.BlockSpec((rb, DIM), lambda i: (i, 0)),
                   pl.BlockSpec((DIM, HDC), lambda i: (0, 0))],
        out_shape=[jax.ShapeDtypeStruct((NTOK, DIM), f32),
                   jax.ShapeDtypeStruct((DIM, HDC), f32)],
        interpret=_INTERPRET,
    )(win_tok, bra_tok, lepe_tok, ca_qkv_w)

    out = pl.pallas_call(
        _ca2_kernel,
        grid=(ngrid,),
        in_specs=[pl.BlockSpec((DIM, HDC), lambda i: (0, 0)),
                  pl.BlockSpec((rb, DIM), lambda i: (i, 0)),
                  pl.BlockSpec((DIM, DIM), lambda i: (0, 0)),
                  pl.BlockSpec((1, DIM), lambda i: (0, 0))],
        out_specs=pl.BlockSpec((rb, DIM), lambda i: (i, 0)),
        out_shape=jax.ShapeDtypeStruct((NTOK, DIM), f32),
        interpret=_INTERPRET,
    )(kv, q_ca, ca_proj_w, ca_proj_b[None, :])

    return out.reshape(1, NTOK, DIM)


# speedup vs baseline: 8.0103x; 3.0788x over previous
import jax
import jax.numpy as jnp
from jax import lax
from jax.experimental import pallas as pl
from jax.experimental.pallas import tpu as pltpu

DIM = 192
NH = 8
WS = 8
TOPK = 4
H = 224
NWH = H // WS
NREG = NWH * NWH
TOK = WS * WS
NTOK = NREG * TOK
CH = DIM // 2
HDW = CH // NH
HDC = DIM // NH
SCALE = DIM ** -0.5
CA_SCALE = HDC ** -0.5

_INTERPRET = False
_STAGE = 0


def _qkv_kernel(x_ref, w_ref, b_ref, owq_ref, owk_ref, owv_ref,
                oq_ref, ok_ref, ov_ref, ol_ref):
    y = jnp.dot(x_ref[...], w_ref[...], preferred_element_type=jnp.float32)
    y = y + b_ref[...]
    owq_ref[...] = y[:, 0:96]
    owk_ref[...] = y[:, 96:192]
    owv_ref[...] = y[:, 192:288]
    oq_ref[...] = y[:, 288:384]
    ok_ref[...] = y[:, 384:480]
    ov_ref[...] = y[:, 480:576]
    ol_ref[...] = jnp.concatenate([y[:, 480:576], y[:, 192:288]], axis=1)


def _softmax_last(x):
    m = jnp.max(x, axis=-1, keepdims=True)
    e = jnp.exp(x - m)
    s = jnp.sum(e, axis=-1, keepdims=True)
    return e * (1.0 / s)


def _win_attn_kernel(q_ref, k_ref, v_ref, mq_ref, am_ref, mo_ref, o_ref):
    q0 = q_ref[:, 0:WS, :].reshape(TOK, 96)
    q1 = q_ref[:, WS:2 * WS, :].reshape(TOK, 96)
    qs = jnp.concatenate([q0] * NH + [q1] * NH, axis=0) * mq_ref[...]
    kc = jnp.concatenate([k_ref[:, 0:WS, :].reshape(TOK, 96),
                          k_ref[:, WS:2 * WS, :].reshape(TOK, 96)], axis=0)
    vc = jnp.concatenate([v_ref[:, 0:WS, :].reshape(TOK, 96),
                          v_ref[:, WS:2 * WS, :].reshape(TOK, 96)], axis=0)
    l = lax.dot_general(qs, kc, (((1,), (1,)), ((), ())),
                        preferred_element_type=jnp.float32)
    l = _softmax_last(l + am_ref[...])
    o = jnp.dot(l, vc, preferred_element_type=jnp.float32)
    for w in range(2):
        acc = o[w * 512:w * 512 + TOK, :] * mo_ref[0:1, :]
        for h in range(1, NH):
            acc = acc + o[w * 512 + h * TOK:w * 512 + (h + 1) * TOK, :] * mo_ref[h:h + 1, :]
        o_ref[:, w * WS:(w + 1) * WS, :] = acc.reshape(WS, WS, 96)


def _pool_kernel(q_ref, k_ref, p_ref, qr_ref, kr_ref):
    p = p_ref[...]
    sq = jnp.sum(q_ref[...], axis=0)
    sk = jnp.sum(k_ref[...], axis=0)
    qr_ref[...] = (jnp.dot(p, sq, preferred_element_type=jnp.float32)
                   * (1.0 / TOK)).reshape(NWH, 1, 96)
    kr_ref[...] = (jnp.dot(p, sk, preferred_element_type=jnp.float32)
                   * (1.0 / TOK)).reshape(NWH, 1, 96)


def _topk_kernel(qr_ref, kr_ref, idx_ref):
    a = lax.dot_general(qr_ref[...], kr_ref[...], (((1,), (1,)), ((), ())),
                        preferred_element_type=jnp.float32)
    col = lax.broadcasted_iota(jnp.int32, a.shape, 1)
    idxs = []
    for _ in range(TOPK):
        m = jnp.max(a, axis=1, keepdims=True)
        sel = jnp.where(a >= m, col, jnp.int32(2 ** 30))
        j = jnp.min(sel, axis=1, keepdims=True)
        idxs.append(j)
        a = jnp.where(col == j, jnp.float32(-3e38), a)
    idx_ref[...] = jnp.concatenate(idxs, axis=1)


def _bra_attn_kernel(idx_ref, q_ref, k0_ref, k1_ref, k2_ref, k3_ref,
                     v0_ref, v1_ref, v2_ref, v3_ref, mq_ref, mo_ref, o_ref):
    del idx_ref
    q = q_ref[...].reshape(TOK, 96)
    qs = jnp.concatenate([q] * NH, axis=0) * mq_ref[...]
    kc = jnp.concatenate([k0_ref[...].reshape(TOK, 96), k1_ref[...].reshape(TOK, 96),
                          k2_ref[...].reshape(TOK, 96), k3_ref[...].reshape(TOK, 96)],
                         axis=0)
    vc = jnp.concatenate([v0_ref[...].reshape(TOK, 96), v1_ref[...].reshape(TOK, 96),
                          v2_ref[...].reshape(TOK, 96), v3_ref[...].reshape(TOK, 96)],
                         axis=0)
    l = lax.dot_general(qs, kc, (((1,), (1,)), ((), ())),
                        preferred_element_type=jnp.float32)
    l = _softmax_last(l)
    o = jnp.dot(l, vc, preferred_element_type=jnp.float32)
    acc = o[0:TOK, :] * mo_ref[0:1, :]
    for h in range(1, NH):
        acc = acc + o[h * TOK:(h + 1) * TOK, :] * mo_ref[h:h + 1, :]
    o_ref[...] = acc.reshape(WS, WS, 96)


def _lepe_kernel(x0_ref, x1_ref, x2_ref, w_ref, b_ref, o_ref):
    acc = jnp.broadcast_to(b_ref[...], (8, H, DIM)).astype(jnp.float32)
    for dy, xref in enumerate((x0_ref, x1_ref, x2_ref)):
        xfull = xref[...]
        for dx in range(3):
            wrow = w_ref[dy * 3 + dx, :]
            acc = acc + xfull[:, dx:dx + H, :] * wrow[None, None, :]
    o_ref[...] = acc


def _ca1_kernel(win_ref, bra_ref, lepe_ref, x_ref, s_ref):
    x = jnp.concatenate([win_ref[...], bra_ref[...]], axis=1) + lepe_ref[...]
    x_ref[...] = x

    @pl.when(pl.program_id(0) == 0)
    def _():
        s_ref[...] = jnp.zeros_like(s_ref)

    s_ref[...] += lax.dot_general(x, x, (((0,), (0,)), ((), ())),
                                  preferred_element_type=jnp.float32)


def _fold_kernel(s_ref, wq_ref, wk_ref, wv_ref, pw_ref, wf_ref):
    t1 = jnp.dot(s_ref[...], wv_ref[...], preferred_element_type=jnp.float32)
    lg = lax.dot_general(wk_ref[...], t1, (((0,), (0,)), ((), ())),
                         preferred_element_type=jnp.float32) * CA_SCALE
    r = lax.broadcasted_iota(jnp.int32, (DIM, DIM), 0) // HDC
    c = lax.broadcasted_iota(jnp.int32, (DIM, DIM), 1) // HDC
    lg = jnp.where(r == c, lg, jnp.float32(-1e30))
    attbd = _softmax_last(lg)
    t2 = lax.dot_general(wq_ref[...], attbd, (((1,), (1,)), ((), ())),
                         preferred_element_type=jnp.float32)
    wf_ref[...] = jnp.dot(t2, pw_ref[...], preferred_element_type=jnp.float32)


def _ca2_kernel(x_ref, wf_ref, pb_ref, o_ref):
    o_ref[...] = (jnp.dot(x_ref[...], wf_ref[...],
                          preferred_element_type=jnp.float32) + pb_ref[...])


def kernel(x, w_win_qkv, b_win_qkv, w_bra_qkv, b_bra_qkv, lepe_w, lepe_b,
           ca_qkv_w, ca_proj_w, ca_proj_b):
    f32 = jnp.float32
    xf = x.reshape(NTOK, DIM)

    z = jnp.zeros((CH, 3 * CH), f32)
    wc = jnp.concatenate([jnp.concatenate([w_win_qkv, z], axis=1),
                          jnp.concatenate([z, w_bra_qkv], axis=1)], axis=0)
    bc = jnp.concatenate([b_win_qkv, b_bra_qkv])[None, :]

    chead = jnp.arange(96, dtype=jnp.int32) // HDW
    hrow = jnp.repeat(jnp.arange(NH, dtype=jnp.int32), TOK)
    maskq = (chead[None, :] == hrow[:, None]).astype(f32) * SCALE
    maskq2 = jnp.concatenate([maskq, maskq], axis=0)
    masko = jax.nn.one_hot(chead, NH, dtype=f32).T
    rgrp = jnp.arange(1024, dtype=jnp.int32) // 512
    cgrp = jnp.arange(128, dtype=jnp.int32) // TOK
    amask = jnp.where(rgrp[:, None] == cgrp[None, :], 0.0, -1e30).astype(f32)
    pmat = jax.nn.one_hot(jnp.arange(H, dtype=jnp.int32) // WS, NWH,
                          dtype=f32).T

    rb = 1792
    ngrid = NTOK // rb
    o_wq, o_wk, o_wv, o_qb, o_kb, o_vb, o_lep = pl.pallas_call(
        _qkv_kernel,
        grid=(ngrid,),
        in_specs=[
            pl.BlockSpec((rb, DIM), lambda i: (i, 0)),
            pl.BlockSpec((DIM, 576), lambda i: (0, 0)),
            pl.BlockSpec((1, 576), lambda i: (0, 0)),
        ],
        out_specs=[pl.BlockSpec((rb, 96), lambda i: (i, 0))] * 6
                  + [pl.BlockSpec((rb, 192), lambda i: (i, 0))],
        out_shape=[jax.ShapeDtypeStruct((NTOK, 96), f32)] * 6
                  + [jax.ShapeDtypeStruct((NTOK, 192), f32)],
        interpret=_INTERPRET,
    )(xf, wc, bc)

    if _STAGE == 1:
        return o_wq
    q_win = o_wq.reshape(H, H, 96)
    k_win = o_wk.reshape(H, H, 96)
    v_win = o_wv.reshape(H, H, 96)
    q_bra = o_qb.reshape(H, H, 96)
    k_bra = o_kb.reshape(H, H, 96)
    v_bra = o_vb.reshape(H, H, 96)

    win_out = pl.pallas_call(
        _win_attn_kernel,
        grid=(NWH, NWH // 2),
        in_specs=[pl.BlockSpec((WS, 2 * WS, 96), lambda i, j: (i, j, 0)),
                  pl.BlockSpec((WS, 2 * WS, 96), lambda i, j: (i, j, 0)),
                  pl.BlockSpec((WS, 2 * WS, 96), lambda i, j: (i, j, 0)),
                  pl.BlockSpec((1024, 96), lambda i, j: (0, 0)),
                  pl.BlockSpec((1024, 128), lambda i, j: (0, 0)),
                  pl.BlockSpec((NH, 96), lambda i, j: (0, 0))],
        out_specs=pl.BlockSpec((WS, 2 * WS, 96), lambda i, j: (i, j, 0)),
        out_shape=jax.ShapeDtypeStruct((H, H, 96), f32),
        interpret=_INTERPRET,
    )(q_win, k_win, v_win, maskq2, amask, masko)

    if _STAGE == 2:
        return win_out

    q_r3, k_r3 = pl.pallas_call(
        _pool_kernel,
        grid=(NWH,),
        in_specs=[pl.BlockSpec((WS, H, 96), lambda i: (i, 0, 0)),
                  pl.BlockSpec((WS, H, 96), lambda i: (i, 0, 0)),
                  pl.BlockSpec((NWH, H), lambda i: (0, 0))],
        out_specs=[pl.BlockSpec((NWH, 1, 96), lambda i: (i, 0, 0)),
                   pl.BlockSpec((NWH, 1, 96), lambda i: (i, 0, 0))],
        out_shape=[jax.ShapeDtypeStruct((NREG, 1, 96), f32),
                   jax.ShapeDtypeStruct((NREG, 1, 96), f32)],
        interpret=_INTERPRET,
    )(q_bra, k_bra, pmat)
    q_r = q_r3.reshape(NREG, 96)
    k_r = k_r3.reshape(NREG, 96)

    pb_ = 112
    idx = pl.pallas_call(
        _topk_kernel,
        grid=(NREG // pb_,),
        in_specs=[pl.BlockSpec((pb_, 96), lambda i: (i, 0)),
                  pl.BlockSpec((NREG, 96), lambda i: (0, 0))],
        out_specs=pl.BlockSpec((pb_, TOPK), lambda i: (i, 0)),
        out_shape=jax.ShapeDtypeStruct((NREG, TOPK), jnp.int32),
        interpret=_INTERPRET,
    )(q_r, k_r)
    idx_flat = idx.reshape(-1)
    if _STAGE == 3:
        return (win_out, idx)

    def _qmap(i, idx_ref):
        return (i // NWH, i % NWH, 0)

    def _gmap(t):
        def m(i, idx_ref):
            r = idx_ref[i * TOPK + t]
            return (r // NWH, r % NWH, 0)
        return m

    bra_out = pl.pallas_call(
        _bra_attn_kernel,
        grid_spec=pltpu.PrefetchScalarGridSpec(
            num_scalar_prefetch=1,
            grid=(NREG,),
            in_specs=[pl.BlockSpec((WS, WS, 96), _qmap)]
                     + [pl.BlockSpec((WS, WS, 96), _gmap(t)) for t in range(TOPK)]
                     + [pl.BlockSpec((WS, WS, 96), _gmap(t)) for t in range(TOPK)]
                     + [pl.BlockSpec((512, 96), lambda i, idx_ref: (0, 0)),
                        pl.BlockSpec((NH, 96), lambda i, idx_ref: (0, 0))],
            out_specs=pl.BlockSpec((WS, WS, 96), _qmap),
        ),
        out_shape=jax.ShapeDtypeStruct((H, H, 96), f32),
        interpret=_INTERPRET,
    )(idx_flat, q_bra, k_bra, k_bra, k_bra, k_bra,
      v_bra, v_bra, v_bra, v_bra, maskq, masko)

    if _STAGE == 4:
        return (win_out, bra_out)

    lepe_hwc = o_lep.reshape(H, H, DIM)
    xp = jnp.pad(lepe_hwc, ((1, 1), (1, 1), (0, 0)))
    x0, x1, x2 = xp[0:H], xp[1:H + 1], xp[2:H + 2]
    wt = jnp.pad(lepe_w.reshape(DIM, 9).T, ((0, 7), (0, 0)))
    lb = lepe_b[None, :]
    lepe = pl.pallas_call(
        _lepe_kernel,
        grid=(H // 8,),
        in_specs=[pl.BlockSpec((8, H + 2, DIM), lambda i: (i, 0, 0)),
                  pl.BlockSpec((8, H + 2, DIM), lambda i: (i, 0, 0)),
                  pl.BlockSpec((8, H + 2, DIM), lambda i: (i, 0, 0)),
                  pl.BlockSpec((16, DIM), lambda i: (0, 0)),
                  pl.BlockSpec((1, DIM), lambda i: (0, 0))],
        out_specs=pl.BlockSpec((8, H, DIM), lambda i: (i, 0, 0)),
        out_shape=jax.ShapeDtypeStruct((H, H, DIM), f32),
        interpret=_INTERPRET,
    )(x0, x1, x2, wt, lb)
    lepe_tok = lepe.reshape(NTOK, DIM)

    if _STAGE == 5:
        return (win_out, bra_out, lepe_tok)

    win_tok = win_out.reshape(NTOK, 96)
    bra_tok = bra_out.reshape(NTOK, 96)

    x_ca, s_acc = pl.pallas_call(
        _ca1_kernel,
        grid=(ngrid,),
        in_specs=[pl.BlockSpec((rb, 96), lambda i: (i, 0)),
                  pl.BlockSpec((rb, 96), lambda i: (i, 0)),
                  pl.BlockSpec((rb, DIM), lambda i: (i, 0))],
        out_specs=[pl.BlockSpec((rb, DIM), lambda i: (i, 0)),
                   pl.BlockSpec((DIM, DIM), lambda i: (0, 0))],
        out_shape=[jax.ShapeDtypeStruct((NTOK, DIM), f32),
                   jax.ShapeDtypeStruct((DIM, DIM), f32)],
        interpret=_INTERPRET,
    )(win_tok, bra_tok, lepe_tok)

    wfold = pl.pallas_call(
        _fold_kernel,
        grid=(1,),
        in_specs=[pl.BlockSpec((DIM, DIM), lambda i: (0, 0))] * 5,
        out_specs=pl.BlockSpec((DIM, DIM), lambda i: (0, 0)),
        out_shape=jax.ShapeDtypeStruct((DIM, DIM), f32),
        interpret=_INTERPRET,
    )(s_acc, ca_qkv_w[:, 0:DIM], ca_qkv_w[:, DIM:2 * DIM],
      ca_qkv_w[:, 2 * DIM:3 * DIM], ca_proj_w)

    out = pl.pallas_call(
        _ca2_kernel,
        grid=(ngrid,),
        in_specs=[pl.BlockSpec((rb, DIM), lambda i: (i, 0)),
                  pl.BlockSpec((DIM, DIM), lambda i: (0, 0)),
                  pl.BlockSpec((1, DIM), lambda i: (0, 0))],
        out_specs=pl.BlockSpec((rb, DIM), lambda i: (i, 0)),
        out_shape=jax.ShapeDtypeStruct((NTOK, DIM), f32),
        interpret=_INTERPRET,
    )(x_ca, wfold, ca_proj_b[None, :])

    return out.reshape(1, NTOK, DIM)
